# baseline (speedup 1.0000x reference)
import functools

import jax
import jax.numpy as jnp
import numpy as np
from jax import lax
from jax.experimental import pallas as pl
from jax.experimental.pallas import tpu as pltpu
from jax.experimental.pallas import tpu_sc as plsc

_P1 = np.int32(np.uint32(2654435761).astype(np.int32))
_P2 = np.int32(805459861)
_RES = 128.0

_NC = 2
_NS = 16
_NW = _NC * _NS
_LANES = 16

_CHUNK = 1024


def _sc_body(*, n_points, mask):
    def body(x_ref, table_ref, out_ref, x_v, w_v, out_v, gsem, *corner_refs):
        idx_refs = corner_refs[0:8]
        row_refs = corner_refs[8:16]
        wid = lax.axis_index("s") * _NC + lax.axis_index("c")
        pts_per_w = n_points // _NW
        n_chunks = pts_per_w // _CHUNK
        iota = lax.iota(jnp.int32, _LANES)
        iota3 = iota * 3
        lane_div4 = lax.shift_right_logical(iota, 2)
        lane_mod4 = jnp.bitwise_and(iota, 3)

        def chunk_body(k, _):
            base = wid * pts_per_w + k * _CHUNK
            pltpu.sync_copy(x_ref.at[pl.ds(base * 3, _CHUNK * 3)], x_v)

            def hash_body(g, _):
                p3 = g * (_LANES * 3) + iota3
                x0 = plsc.load_gather(x_v, [p3])
                x1 = plsc.load_gather(x_v, [p3 + 1])
                x2 = plsc.load_gather(x_v, [p3 + 2])
                s0 = x0 * _RES
                s1 = x1 * _RES
                s2 = x2 * _RES
                f0 = s0.astype(jnp.int32)
                f1 = s1.astype(jnp.int32)
                f2 = s2.astype(jnp.int32)
                f0f = f0.astype(jnp.float32)
                f1f = f1.astype(jnp.float32)
                f2f = f2.astype(jnp.float32)
                d0 = s0 - f0f
                d1 = s1 - f1f
                d2 = s2 - f2f
                c0 = f0 + (s0 > f0f).astype(jnp.int32)
                c1 = f1 + (s1 > f1f).astype(jnp.int32)
                c2 = f2 + (s2 > f2f).astype(jnp.int32)
                bf = f1 * _P1
                bc = c1 * _P1
                gf = f2 * _P2
                gc = c2 * _P2
                t00 = f0 ^ bf
                t10 = c0 ^ bf
                t01 = f0 ^ bc
                t11 = c0 ^ bc
                sl = pl.ds(g * _LANES, _LANES)
                idx_refs[0][sl] = (t00 ^ gf) & mask
                idx_refs[1][sl] = (t10 ^ gf) & mask
                idx_refs[2][sl] = (t01 ^ gf) & mask
                idx_refs[3][sl] = (t00 ^ gc) & mask
                idx_refs[4][sl] = (t11 ^ gf) & mask
                idx_refs[5][sl] = (t10 ^ gc) & mask
                idx_refs[6][sl] = (t01 ^ gc) & mask
                idx_refs[7][sl] = (t11 ^ gc) & mask
                u0 = 1.0 - d0
                u1 = 1.0 - d1
                u2 = 1.0 - d2
                a00 = u0 * u1
                a10 = d0 * u1
                a01 = u0 * d1
                a11 = d0 * d1
                w_v[0, sl] = a00 * u2
                w_v[1, sl] = a10 * u2
                w_v[2, sl] = a01 * u2
                w_v[3, sl] = a00 * d2
                w_v[4, sl] = a11 * u2
                w_v[5, sl] = a10 * d2
                w_v[6, sl] = a01 * d2
                w_v[7, sl] = a11 * d2
                return _

            lax.fori_loop(0, _CHUNK // _LANES, hash_body, None)

            copies = [
                pltpu.async_copy(table_ref.at[idx_refs[c]], row_refs[c], gsem)
                for c in range(8)
            ]
            for cp in copies:
                cp.wait()

            def interp_body(j, _):
                e = j * 4 + lane_div4
                acc = jnp.zeros((_LANES,), jnp.float32)
                for c in range(8):
                    csp = jnp.full((_LANES,), c, jnp.int32)
                    w = plsc.load_gather(w_v, [csp, e])
                    r = plsc.load_gather(row_refs[c], [e, lane_mod4])
                    acc = acc + w * r
                out_v[pl.ds(j * (_LANES), _LANES)] = acc
                return _

            lax.fori_loop(0, _CHUNK // 4, interp_body, None)

            pltpu.sync_copy(out_v, out_ref.at[pl.ds(base * 4, _CHUNK * 4)])
            return _

        lax.fori_loop(0, n_chunks, chunk_body, None)

    return body


@functools.partial(jax.jit, static_argnames=())
def _run(x_flat, hashtable):
    n_points = x_flat.shape[0] // 3
    size = hashtable.shape[0]
    mask = np.int32(size - 1)
    mesh = plsc.VectorSubcoreMesh(
        core_axis_name="c", subcore_axis_name="s", num_cores=_NC, num_subcores=_NS
    )
    body = _sc_body(n_points=n_points, mask=mask)
    fn = pl.kernel(
        body,
        out_type=jax.ShapeDtypeStruct((n_points * 4,), jnp.float32),
        mesh=mesh,
        scratch_types=[
            pltpu.VMEM((_CHUNK * 3,), jnp.float32),
            pltpu.VMEM((8, _CHUNK), jnp.float32),
            pltpu.VMEM((_CHUNK * 4,), jnp.float32),
            pltpu.SemaphoreType.DMA,
        ]
        + [pltpu.VMEM((_CHUNK,), jnp.int32) for _ in range(8)]
        + [pltpu.VMEM((_CHUNK, 8), jnp.float32) for _ in range(8)],
        compiler_params=pltpu.CompilerParams(
            needs_layout_passes=False, use_tc_tiling_on_sc=False
        ),
    )
    return fn(x_flat, hashtable)


def kernel(x, hashtable):
    table8 = jnp.pad(hashtable, ((0, 0), (0, 8 - hashtable.shape[1])))
    out_flat = _run(x.reshape(-1), table8)
    return out_flat.reshape(x.shape[:-1] + (4,))
